# Initial kernel scaffold; baseline (speedup 1.0000x reference)
import jax
import jax.numpy as jnp
from jax.experimental import pallas as pl


def kernel(hidden_states, wg, We, be):
    raise NotImplementedError("write your pallas kernel here")


# speedup vs baseline: 1.6571x; 1.6571x over previous
import functools
import math

import jax
import jax.numpy as jnp
import numpy as np
from jax import lax
from jax.experimental import pallas as pl
from jax.experimental.pallas import tpu as pltpu
from jax.experimental.pallas import tpu_sc as plsc

_SC_CORES = 2
_SC_SUBCORES = 16
_LANES = 16
_NWORKERS = _SC_CORES * _SC_SUBCORES

_TOK_BLK = 512
_CHUNK = 128
_GW = 128


def _router_body(nblk, cap, n_tok, trash, x_ref, wg_ref,
                 disp_ref, grep_ref, laux_ref, cnts_ref,
                 counts_sc, me_sc):
    i = pl.program_id(0)
    e = wg_ref.shape[1]

    @pl.when(i == 0)
    def _init():
        counts_sc[...] = jnp.zeros_like(counts_sc)
        me_sc[...] = jnp.zeros_like(me_sc)

    x = x_ref[...]
    logits = jnp.dot(x, wg_ref[...], preferred_element_type=jnp.float32)
    mx = jnp.max(logits, axis=1, keepdims=True)
    ex = jnp.exp(logits - mx)
    gates = ex / jnp.sum(ex, axis=1, keepdims=True)
    gmax = jnp.max(gates, axis=1, keepdims=True)
    cols = lax.broadcasted_iota(jnp.int32, gates.shape, 1)
    idx1 = jnp.min(jnp.where(gates >= gmax, cols, e), axis=1)
    onehot = (cols == idx1[:, None]).astype(jnp.float32)

    blk = x.shape[0]
    r = lax.broadcasted_iota(jnp.int32, (blk, blk), 0)
    c = lax.broadcasted_iota(jnp.int32, (blk, blk), 1)
    ltri = (c < r).astype(jnp.bfloat16)
    loc_excl = jnp.dot(ltri, onehot.astype(jnp.bfloat16),
                       preferred_element_type=jnp.float32)
    loc = loc_excl + counts_sc[0, :][None, :]
    loc_t = jnp.sum(loc * onehot, axis=1)
    kept = loc_t < float(cap)
    flat = idx1 * cap + loc_t.astype(jnp.int32)
    disp_ref[0, 0, :] = jnp.where(kept, flat, trash)
    g1 = gmax[:, 0] * kept.astype(jnp.float32)
    grep_ref[...] = jnp.broadcast_to(g1[:, None], (blk, _GW))

    counts_sc[0, :] += jnp.sum(onehot, axis=0)
    me_sc[0, :] += jnp.sum(gates, axis=0)

    @pl.when(i == nblk - 1)
    def _fin():
        cnt = counts_sc[0, :]
        laux = jnp.sum((me_sc[0, :] / n_tok) * (cnt / n_tok)) * float(e)
        laux_ref[0, :] = jnp.broadcast_to(laux, (e,))
        cnts_ref[0, :] = cnt


def _run_router(x, wg, cap, trash):
    n_tok, d = x.shape
    e = wg.shape[1]
    nblk = n_tok // _TOK_BLK
    disp3, grep, laux, cnts = pl.pallas_call(
        functools.partial(_router_body, nblk, cap, float(n_tok), trash),
        grid=(nblk,),
        in_specs=[
            pl.BlockSpec((_TOK_BLK, d), lambda i: (i, 0)),
            pl.BlockSpec((d, e), lambda i: (0, 0)),
        ],
        out_specs=[
            pl.BlockSpec((1, 1, _TOK_BLK), lambda i: (i, 0, 0)),
            pl.BlockSpec((_TOK_BLK, _GW), lambda i: (i, 0)),
            pl.BlockSpec((1, e), lambda i: (0, 0)),
            pl.BlockSpec((1, e), lambda i: (0, 0)),
        ],
        out_shape=[
            jax.ShapeDtypeStruct((nblk, 1, _TOK_BLK), jnp.int32),
            jax.ShapeDtypeStruct((n_tok, _GW), jnp.float32),
            jax.ShapeDtypeStruct((1, e), jnp.float32),
            jax.ShapeDtypeStruct((1, e), jnp.float32),
        ],
        scratch_shapes=[
            pltpu.VMEM((1, e), jnp.float32),
            pltpu.VMEM((1, e), jnp.float32),
        ],
    )(x, wg)
    return disp3.reshape(n_tok), grep, laux, cnts


def _make_dispatch(n_tok, d, nslot):
    per_w = n_tok // _NWORKERS
    nchunk = per_w // _CHUNK
    mesh = plsc.VectorSubcoreMesh(core_axis_name="c", subcore_axis_name="s")

    @functools.partial(
        pl.kernel, mesh=mesh,
        out_type=[
            jax.ShapeDtypeStruct((nslot, d), jnp.float32),
            jax.ShapeDtypeStruct((nslot, _GW), jnp.float32),
        ],
        scratch_types=[
            pltpu.VMEM((_CHUNK,), jnp.int32),
            pltpu.VMEM((_CHUNK, d), jnp.float32),
            pltpu.VMEM((_CHUNK, _GW), jnp.float32),
            pltpu.SemaphoreType.DMA,
        ],
    )
    def disp_kernel(x_hbm, grep_hbm, idx_hbm, out_hbm, sg_hbm,
                    idx_v, rows_v, g_v, sem):
        wid = lax.axis_index("s") * _SC_CORES + lax.axis_index("c")
        for j in range(nchunk):
            base = wid * per_w + j * _CHUNK
            pltpu.sync_copy(idx_hbm.at[pl.ds(base, _CHUNK)], idx_v)
            pltpu.sync_copy(x_hbm.at[pl.ds(base, _CHUNK)], rows_v)
            pltpu.sync_copy(grep_hbm.at[pl.ds(base, _CHUNK)], g_v)
            c1 = pltpu.async_copy(rows_v, out_hbm.at[idx_v], sem)
            c1.wait()
            c2 = pltpu.async_copy(g_v, sg_hbm.at[idx_v], sem)
            c2.wait()

    return disp_kernel


def _ffn_body(e_num, disp_ref, we_ref, be_ref, sg_ref, out_ref):
    e = pl.program_id(0)

    @pl.when(e < e_num)
    def _():
        acc = jnp.dot(disp_ref[...], we_ref[0],
                      preferred_element_type=jnp.float32)
        acc = acc + be_ref[0, 0, :][None, :]
        out_ref[...] = acc * sg_ref[:, 0:1]

    @pl.when(e == e_num)
    def _():
        out_ref[...] = jnp.zeros_like(out_ref)


def _run_ffn(dispd, we, be, slot_g, cap):
    e_num, d = we.shape[0], we.shape[1]
    nslot = dispd.shape[0]
    return pl.pallas_call(
        functools.partial(_ffn_body, e_num),
        grid=(nslot // cap,),
        in_specs=[
            pl.BlockSpec((cap, d), lambda e: (e, 0)),
            pl.BlockSpec((1, d, d), lambda e: (jnp.minimum(e, e_num - 1), 0, 0)),
            pl.BlockSpec((1, 1, d), lambda e: (jnp.minimum(e, e_num - 1), 0, 0)),
            pl.BlockSpec((cap, _GW), lambda e: (e, 0)),
        ],
        out_specs=pl.BlockSpec((cap, d), lambda e: (e, 0)),
        out_shape=jax.ShapeDtypeStruct((nslot, d), jnp.float32),
    )(dispd, we, be.reshape(e_num, 1, d), slot_g)


def _make_combine(n_tok, d, nslot):
    per_w = n_tok // _NWORKERS
    nchunk = per_w // _CHUNK
    mesh = plsc.VectorSubcoreMesh(core_axis_name="c", subcore_axis_name="s")

    @functools.partial(
        pl.kernel, mesh=mesh,
        out_type=jax.ShapeDtypeStruct((n_tok, d), jnp.float32),
        scratch_types=[
            pltpu.VMEM((_CHUNK,), jnp.int32),
            pltpu.VMEM((_CHUNK, d), jnp.float32),
            pltpu.SemaphoreType.DMA,
        ],
    )
    def comb_kernel(eo_hbm, idx_hbm, out_hbm, idx_v, rows_v, sem):
        wid = lax.axis_index("s") * _SC_CORES + lax.axis_index("c")
        for j in range(nchunk):
            base = wid * per_w + j * _CHUNK
            pltpu.sync_copy(idx_hbm.at[pl.ds(base, _CHUNK)], idx_v)
            c = pltpu.async_copy(eo_hbm.at[idx_v], rows_v, sem)
            c.wait()
            pltpu.sync_copy(rows_v, out_hbm.at[pl.ds(base, _CHUNK)])

    return comb_kernel


def kernel(hidden_states, wg, We, be):
    b, s, d = hidden_states.shape
    x = hidden_states.reshape(-1, d)
    n_tok = b * s
    e = wg.shape[1]
    cap = max(int(math.ceil(n_tok / e)), 4)
    trash = e * cap
    nslot = e * cap + cap

    disp_idx, grep, laux, cnts = _run_router(x, wg, cap, trash)
    dispd, slot_g = _make_dispatch(n_tok, d, nslot)(x, grep, disp_idx)
    eo = _run_ffn(dispd, We, be, slot_g, cap)
    out = _make_combine(n_tok, d, nslot)(eo, disp_idx)

    return (out.reshape(b, s, d), laux[0, 0], cnts[0, :].astype(jnp.int32))


# speedup vs baseline: 1.6770x; 1.0120x over previous
import functools
import math

import jax
import jax.numpy as jnp
import numpy as np
from jax import lax
from jax.experimental import pallas as pl
from jax.experimental.pallas import tpu as pltpu
from jax.experimental.pallas import tpu_sc as plsc

_SC_CORES = 2
_SC_SUBCORES = 16
_LANES = 16
_NWORKERS = _SC_CORES * _SC_SUBCORES

_TOK_BLK = 512
_CHUNK = 64
_GW = 128


def _router_body(nblk, cap, n_tok, trash, x_ref, wg_ref,
                 disp_ref, grep_ref, laux_ref, cnts_ref,
                 counts_sc, me_sc):
    i = pl.program_id(0)
    e = wg_ref.shape[1]

    @pl.when(i == 0)
    def _init():
        counts_sc[...] = jnp.zeros_like(counts_sc)
        me_sc[...] = jnp.zeros_like(me_sc)

    x = x_ref[...]
    logits = jnp.dot(x, wg_ref[...], preferred_element_type=jnp.float32)
    mx = jnp.max(logits, axis=1, keepdims=True)
    ex = jnp.exp(logits - mx)
    gates = ex / jnp.sum(ex, axis=1, keepdims=True)
    gmax = jnp.max(gates, axis=1, keepdims=True)
    cols = lax.broadcasted_iota(jnp.int32, gates.shape, 1)
    idx1 = jnp.min(jnp.where(gates >= gmax, cols, e), axis=1)
    onehot = (cols == idx1[:, None]).astype(jnp.float32)

    blk = x.shape[0]
    r = lax.broadcasted_iota(jnp.int32, (blk, blk), 0)
    c = lax.broadcasted_iota(jnp.int32, (blk, blk), 1)
    ltri = (c < r).astype(jnp.bfloat16)
    loc_excl = jnp.dot(ltri, onehot.astype(jnp.bfloat16),
                       preferred_element_type=jnp.float32)
    loc = loc_excl + counts_sc[0, :][None, :]
    loc_t = jnp.sum(loc * onehot, axis=1)
    kept = loc_t < float(cap)
    flat = idx1 * cap + loc_t.astype(jnp.int32)
    disp_ref[0, 0, :] = jnp.where(kept, flat, trash)
    g1 = gmax[:, 0] * kept.astype(jnp.float32)
    grep_ref[...] = jnp.broadcast_to(g1[:, None], (blk, _GW))

    counts_sc[0, :] += jnp.sum(onehot, axis=0)
    me_sc[0, :] += jnp.sum(gates, axis=0)

    @pl.when(i == nblk - 1)
    def _fin():
        cnt = counts_sc[0, :]
        laux = jnp.sum((me_sc[0, :] / n_tok) * (cnt / n_tok)) * float(e)
        laux_ref[0, :] = jnp.broadcast_to(laux, (e,))
        cnts_ref[0, :] = cnt


def _run_router(x, wg, cap, trash):
    n_tok, d = x.shape
    e = wg.shape[1]
    nblk = n_tok // _TOK_BLK
    disp3, grep, laux, cnts = pl.pallas_call(
        functools.partial(_router_body, nblk, cap, float(n_tok), trash),
        grid=(nblk,),
        in_specs=[
            pl.BlockSpec((_TOK_BLK, d), lambda i: (i, 0)),
            pl.BlockSpec((d, e), lambda i: (0, 0)),
        ],
        out_specs=[
            pl.BlockSpec((1, 1, _TOK_BLK), lambda i: (i, 0, 0)),
            pl.BlockSpec((_TOK_BLK, _GW), lambda i: (i, 0)),
            pl.BlockSpec((1, e), lambda i: (0, 0)),
            pl.BlockSpec((1, e), lambda i: (0, 0)),
        ],
        out_shape=[
            jax.ShapeDtypeStruct((nblk, 1, _TOK_BLK), jnp.int32),
            jax.ShapeDtypeStruct((n_tok, _GW), jnp.float32),
            jax.ShapeDtypeStruct((1, e), jnp.float32),
            jax.ShapeDtypeStruct((1, e), jnp.float32),
        ],
        scratch_shapes=[
            pltpu.VMEM((1, e), jnp.float32),
            pltpu.VMEM((1, e), jnp.float32),
        ],
    )(x, wg)
    return disp3.reshape(n_tok), grep, laux, cnts


def _make_dispatch(n_tok, d, nslot):
    per_w = n_tok // _NWORKERS
    nchunk = per_w // _CHUNK
    mesh = plsc.VectorSubcoreMesh(core_axis_name="c", subcore_axis_name="s")

    @functools.partial(
        pl.kernel, mesh=mesh,
        out_type=[
            jax.ShapeDtypeStruct((nslot, d), jnp.float32),
            jax.ShapeDtypeStruct((nslot, _GW), jnp.float32),
        ],
        scratch_types=[
            pltpu.VMEM((nchunk, _CHUNK), jnp.int32),
            pltpu.VMEM((2, _CHUNK, d), jnp.float32),
            pltpu.VMEM((2, _CHUNK, _GW), jnp.float32),
            pltpu.SemaphoreType.DMA,
            pltpu.SemaphoreType.DMA,
            pltpu.SemaphoreType.DMA,
            pltpu.SemaphoreType.DMA,
        ],
    )
    def disp_kernel(x_hbm, grep_hbm, idx2_hbm, out_hbm, sg_hbm,
                    idx_v, rows_v, g_v, sl0, sl1, ss0, ss1):
        wid = lax.axis_index("s") * _SC_CORES + lax.axis_index("c")
        seml, sems = (sl0, sl1), (ss0, ss1)
        pltpu.sync_copy(idx2_hbm.at[pl.ds(wid * nchunk, nchunk)], idx_v)

        def fire_loads(j, b):
            tb = wid * per_w + j * _CHUNK
            return (
                pltpu.async_copy(x_hbm.at[pl.ds(tb, _CHUNK)],
                                 rows_v.at[b], seml[b]),
                pltpu.async_copy(grep_hbm.at[pl.ds(tb, _CHUNK)],
                                 g_v.at[b], seml[b]),
            )

        def fire_scats(j, b):
            return (
                pltpu.async_copy(rows_v.at[b], out_hbm.at[idx_v.at[j]],
                                 sems[b]),
                pltpu.async_copy(g_v.at[b], sg_hbm.at[idx_v.at[j]],
                                 sems[b]),
            )

        loads = {0: fire_loads(0, 0)}
        scats = {}
        for j in range(nchunk):
            b = j & 1
            if j + 1 < nchunk:
                if j - 1 in scats:
                    for c in scats.pop(j - 1):
                        c.wait()
                loads[j + 1] = fire_loads(j + 1, 1 - b)
            for c in loads.pop(j):
                c.wait()
            scats[j] = fire_scats(j, b)
        for jj in sorted(scats):
            for c in scats.pop(jj):
                c.wait()

    return disp_kernel


def _ffn_body(e_num, disp_ref, we_ref, be_ref, sg_ref, out_ref):
    e = pl.program_id(0)

    @pl.when(e < e_num)
    def _():
        acc = jnp.dot(disp_ref[...].astype(jnp.bfloat16),
                      we_ref[0].astype(jnp.bfloat16),
                      preferred_element_type=jnp.float32)
        acc = acc + be_ref[0, 0, :][None, :]
        out_ref[...] = acc * sg_ref[:, 0:1]

    @pl.when(e == e_num)
    def _():
        out_ref[...] = jnp.zeros_like(out_ref)


def _run_ffn(dispd, we, be, slot_g, cap):
    e_num, d = we.shape[0], we.shape[1]
    nslot = dispd.shape[0]
    return pl.pallas_call(
        functools.partial(_ffn_body, e_num),
        grid=(nslot // cap,),
        in_specs=[
            pl.BlockSpec((cap, d), lambda e: (e, 0)),
            pl.BlockSpec((1, d, d), lambda e: (jnp.minimum(e, e_num - 1), 0, 0)),
            pl.BlockSpec((1, 1, d), lambda e: (jnp.minimum(e, e_num - 1), 0, 0)),
            pl.BlockSpec((cap, _GW), lambda e: (e, 0)),
        ],
        out_specs=pl.BlockSpec((cap, d), lambda e: (e, 0)),
        out_shape=jax.ShapeDtypeStruct((nslot, d), jnp.float32),
    )(dispd, we, be.reshape(e_num, 1, d), slot_g)


def _make_combine(n_tok, d, nslot):
    per_w = n_tok // _NWORKERS
    nchunk = per_w // _CHUNK
    mesh = plsc.VectorSubcoreMesh(core_axis_name="c", subcore_axis_name="s")

    @functools.partial(
        pl.kernel, mesh=mesh,
        out_type=jax.ShapeDtypeStruct((n_tok, d), jnp.float32),
        scratch_types=[
            pltpu.VMEM((nchunk, _CHUNK), jnp.int32),
            pltpu.VMEM((2, _CHUNK, d), jnp.float32),
            pltpu.SemaphoreType.DMA,
            pltpu.SemaphoreType.DMA,
            pltpu.SemaphoreType.DMA,
            pltpu.SemaphoreType.DMA,
        ],
    )
    def comb_kernel(eo_hbm, idx2_hbm, out_hbm, idx_v, rows_v,
                    sg0, sg1, ss0, ss1):
        wid = lax.axis_index("s") * _SC_CORES + lax.axis_index("c")
        semg, sems = (sg0, sg1), (ss0, ss1)
        pltpu.sync_copy(idx2_hbm.at[pl.ds(wid * nchunk, nchunk)], idx_v)

        def fire_gather(j, b):
            return pltpu.async_copy(eo_hbm.at[idx_v.at[j]], rows_v.at[b],
                                    semg[b])

        def fire_store(j, b):
            tb = wid * per_w + j * _CHUNK
            return pltpu.async_copy(rows_v.at[b],
                                    out_hbm.at[pl.ds(tb, _CHUNK)], sems[b])

        gath = {0: fire_gather(0, 0)}
        stor = {}
        for j in range(nchunk):
            b = j & 1
            gath.pop(j).wait()
            if j + 1 < nchunk:
                if j - 1 in stor:
                    stor.pop(j - 1).wait()
                gath[j + 1] = fire_gather(j + 1, 1 - b)
            stor[j] = fire_store(j, b)
        for jj in sorted(stor):
            stor.pop(jj).wait()

    return comb_kernel


def kernel(hidden_states, wg, We, be):
    b, s, d = hidden_states.shape
    x = hidden_states.reshape(-1, d)
    n_tok = b * s
    e = wg.shape[1]
    cap = max(int(math.ceil(n_tok / e)), 4)
    trash = e * cap
    nslot = e * cap + cap

    disp_idx, grep, laux, cnts = _run_router(x, wg, cap, trash)
    idx2 = disp_idx.reshape(n_tok // _CHUNK, _CHUNK)
    dispd, slot_g = _make_dispatch(n_tok, d, nslot)(x, grep, idx2)
    eo = _run_ffn(dispd, We, be, slot_g, cap)
    out = _make_combine(n_tok, d, nslot)(eo, idx2)

    return (out.reshape(b, s, d), laux[0, 0], cnts[0, :].astype(jnp.int32))
